# Initial kernel scaffold; baseline (speedup 1.0000x reference)
import jax
import jax.numpy as jnp
from jax.experimental import pallas as pl


def kernel(tokens, table):
    raise NotImplementedError("write your pallas kernel here")


# speedup vs baseline: 6.8130x; 6.8130x over previous
import math

import jax
import jax.numpy as jnp
from jax.experimental import pallas as pl
from jax.experimental.pallas import tpu as pltpu
from jax.experimental.pallas import tpu_sc as plsc

_WINDOW = 128


def _scale_table(table, scale):
    v, d = table.shape
    rows = 4000
    assert v % rows == 0

    def body(t_ref, o_ref):
        o_ref[...] = t_ref[...] * scale

    return pl.pallas_call(
        body,
        grid=(v // rows,),
        in_specs=[pl.BlockSpec((rows, d), lambda i: (i, 0))],
        out_specs=pl.BlockSpec((rows, d), lambda i: (i, 0)),
        out_shape=jax.ShapeDtypeStruct((v, d), table.dtype),
    )(table)


def kernel(tokens, table):
    v, d = table.shape
    b, s = tokens.shape
    n = b * s
    assert n % _WINDOW == 0

    scaled = _scale_table(table, math.sqrt(d))
    idx = tokens.reshape(1, n).astype(jnp.int32)

    mesh = plsc.VectorSubcoreMesh(
        core_axis_name="core", subcore_axis_name="subcore"
    )

    @pl.kernel(
        out_type=jax.ShapeDtypeStruct((n, d), table.dtype),
        mesh=mesh,
    )
    def gather_kernel(x_hbm, i_hbm, o_hbm):
        def body(i_vmem, o_vmem):
            pltpu.sync_copy(x_hbm.at[i_vmem.at[0]], o_vmem)

        pltpu.emit_pipeline(
            body,
            grid=(n // _WINDOW,),
            in_specs=[pl.BlockSpec((1, _WINDOW), index_map=lambda i: (0, i))],
            out_specs=[pl.BlockSpec((_WINDOW, d), index_map=lambda i: (i, 0))],
            core_axis_name=("core", "subcore"),
            dimension_semantics=(pltpu.PARALLEL,),
        )(i_hbm, o_hbm)

    out = gather_kernel(scaled, idx)
    return out.reshape(b, s, d)


# speedup vs baseline: 8.2298x; 1.2080x over previous
import math

import jax
import jax.numpy as jnp
from jax import lax
from jax.experimental import pallas as pl
from jax.experimental.pallas import tpu as pltpu
from jax.experimental.pallas import tpu_sc as plsc

_NW = 32
_WIN = 128
_W = 2
_K = _W * _WIN


def _scale_table(table, scale):
    v, d = table.shape
    rows = 4000
    assert v % rows == 0

    def body(t_ref, o_ref):
        o_ref[...] = t_ref[...] * scale

    return pl.pallas_call(
        body,
        grid=(v // rows,),
        in_specs=[pl.BlockSpec((rows, d), lambda i: (i, 0))],
        out_specs=pl.BlockSpec((rows, d), lambda i: (i, 0)),
        out_shape=jax.ShapeDtypeStruct((v, d), table.dtype),
    )(table)


def kernel(tokens, table):
    v, d = table.shape
    b, s = tokens.shape
    n = b * s
    assert n % (_NW * _K) == 0
    c_per_w = n // (_NW * _K)
    assert c_per_w % 2 == 0

    scaled = _scale_table(table, math.sqrt(d))
    idx = tokens.reshape(_NW, c_per_w, _W, _WIN).astype(jnp.int32)

    mesh = plsc.VectorSubcoreMesh(
        core_axis_name="core", subcore_axis_name="subcore"
    )

    @pl.kernel(
        out_type=jax.ShapeDtypeStruct((_NW, c_per_w, _K, d), table.dtype),
        mesh=mesh,
        scratch_types=[
            pltpu.VMEM((2, _W, _WIN), jnp.int32),
            pltpu.VMEM((2, _K, d), jnp.float32),
            pltpu.SemaphoreType.DMA((2,)),
            pltpu.SemaphoreType.DMA((2,)),
            pltpu.SemaphoreType.DMA((2,)),
        ],
    )
    def gather_kernel(x_hbm, i_hbm, o_hbm, idx_v, rows_v, gsem, wsem, isem):
        wid = lax.axis_index("subcore") * 2 + lax.axis_index("core")

        def fire_gathers(bb, cc):
            for j in range(_W):
                pltpu.make_async_copy(
                    x_hbm.at[idx_v.at[bb, j]],
                    rows_v.at[bb, pl.ds(j * _WIN, _WIN)],
                    gsem.at[bb],
                ).start()

        pltpu.sync_copy(i_hbm.at[wid, 0], idx_v.at[0])
        fire_gathers(0, 0)
        pltpu.make_async_copy(i_hbm.at[wid, 1], idx_v.at[1], isem.at[1]).start()

        def work(c, bb, nb):
            @pl.when(c + 1 < c_per_w)
            def _():
                pltpu.make_async_copy(
                    i_hbm.at[wid, 0], idx_v.at[nb], isem.at[nb]
                ).wait()

                @pl.when(c > 0)
                def _():
                    pltpu.make_async_copy(
                        rows_v.at[nb], o_hbm.at[wid, 0], wsem.at[nb]
                    ).wait()

                fire_gathers(nb, c + 1)

            pltpu.make_async_copy(
                o_hbm.at[wid, 0], rows_v.at[bb], gsem.at[bb]
            ).wait()

            @pl.when(c + 2 < c_per_w)
            def _():
                pltpu.make_async_copy(
                    i_hbm.at[wid, c + 2], idx_v.at[bb], isem.at[bb]
                ).start()

            pltpu.make_async_copy(
                rows_v.at[bb], o_hbm.at[wid, c], wsem.at[bb]
            ).start()

        @pl.loop(0, c_per_w, step=2)
        def _(c):
            work(c, 0, 1)
            work(c + 1, 1, 0)

        pltpu.make_async_copy(
            rows_v.at[0], o_hbm.at[wid, 0], wsem.at[0]
        ).wait()
        pltpu.make_async_copy(
            rows_v.at[1], o_hbm.at[wid, 0], wsem.at[1]
        ).wait()

    out = gather_kernel(scaled, idx)
    return out.reshape(b, s, d)
